# baseline (speedup 1.0000x reference)
import functools

import jax
import jax.numpy as jnp
from jax import lax
from jax.experimental import pallas as pl
from jax.experimental.pallas import tpu as pltpu
from jax.experimental.pallas import tpu_sc as plsc

N = 16384
D = 128
LEVELS = 4
NC = 2
NS = 16
NW = NC * NS
ROWS_PER_W = N // NW
CHUNK = 64
NCHUNK = ROWS_PER_W // CHUNK
LANES = 16
NSETS = 3
GROWS = LEVELS * CHUNK


def _body(idx_hbm, table, out, idx_v, b0, b1, b2, o0, o1, o2, sems):
    wid = lax.axis_index("s") * NC + lax.axis_index("c")
    base = wid * ROWS_PER_W
    bufs = (b0, b1, b2)
    obufs = (o0, o1, o2)
    gsems, wsems = sems[:NSETS], sems[NSETS:]

    pltpu.sync_copy(idx_hbm.at[wid], idx_v)

    def fire_gathers(k, s):
        return [
            pltpu.async_copy(table.at[idx_v.at[2 * k + j]],
                             bufs[s].at[pl.ds(j * 2 * CHUNK, 2 * CHUNK)],
                             gsems[s])
            for j in range(2)
        ]

    gcps = [fire_gathers(0, 0), fire_gathers(1, 1), None]
    wcps = [None] * NSETS
    for k in range(NCHUNK):
        s = k % NSETS
        if k + 2 < NCHUNK:
            gcps[(k + 2) % NSETS] = fire_gathers(k + 2, (k + 2) % NSETS)
        with jax.named_scope("gwait"):
            for cp in gcps[s]:
                cp.wait()
        with jax.named_scope("wbwait"):
            if wcps[s] is not None:
                wcps[s].wait()
        b, o = bufs[s], obufs[s]

        def add_row(r, _, b=b, o=o):
            m = r * LEVELS
            for col in range(D // LANES):
                sl = pl.ds(col * LANES, LANES)
                o[r, sl] = (b[m, sl] + b[m + 1, sl]
                            + b[m + 2, sl] + b[m + 3, sl])
            return 0

        with jax.named_scope("adds"):
            lax.fori_loop(0, CHUNK, add_row, 0)
        wcps[s] = pltpu.async_copy(
            o, out.at[pl.ds(base + k * CHUNK, CHUNK)], wsems[s])
    for cp in wcps:
        if cp is not None:
            cp.wait()


def _entry(idx_hbm, table, out, idx_v, b0, b1, b2, o0, o1, o2, *sems):
    _body(idx_hbm, table, out, idx_v, b0, b1, b2, o0, o1, o2, sems)


_mesh = plsc.VectorSubcoreMesh(core_axis_name="c", subcore_axis_name="s")

_sc_call = functools.partial(
    pl.kernel,
    mesh=_mesh,
    out_type=jax.ShapeDtypeStruct((N, D), jnp.float32),
    scratch_types=(
        [pltpu.VMEM((2 * NCHUNK, 2 * CHUNK), jnp.int32)]
        + [pltpu.VMEM((GROWS, D), jnp.float32)] * NSETS
        + [pltpu.VMEM((CHUNK, D), jnp.float32)] * NSETS
        + [pltpu.SemaphoreType.DMA] * (2 * NSETS)
    ),
)(_entry)


@jax.jit
def kernel(coords, emb0, emb1, emb2, emb3):
    table = jnp.stack([emb0, emb1, emb2, emb3], axis=1).reshape(-1, D)
    idx = coords * LEVELS + jnp.arange(LEVELS, dtype=jnp.int32)
    idx = idx.reshape(NW, 2 * NCHUNK, 2 * CHUNK)
    return _sc_call(idx, table)


# speedup vs baseline: 1.6876x; 1.6876x over previous
import functools

import jax
import jax.numpy as jnp
from jax import lax
from jax.experimental import pallas as pl
from jax.experimental.pallas import tpu as pltpu
from jax.experimental.pallas import tpu_sc as plsc

N = 16384
D = 128
LEVELS = 4
NC = 2
NS = 16
NW = NC * NS
ROWS_PER_W = N // NW
CHUNK = 64
NCHUNK = ROWS_PER_W // CHUNK
LANES = 16
NSETS = 3
GROWS = LEVELS * CHUNK


def _body(idx_hbm, table, out, idx_v, b0, b1, b2, o0, o1, o2, sems):
    wid = lax.axis_index("s") * NC + lax.axis_index("c")
    base = wid * ROWS_PER_W
    bufs = (b0, b1, b2)
    obufs = (o0, o1, o2)
    gsems, wsems = sems[:NSETS], sems[NSETS:]

    pltpu.sync_copy(idx_hbm.at[wid], idx_v)

    def fire_gathers(k, s):
        return [
            pltpu.async_copy(table.at[idx_v.at[2 * k + j]],
                             bufs[s].at[pl.ds(j * 2 * CHUNK, 2 * CHUNK)],
                             gsems[s])
            for j in range(2)
        ]

    gcps = [fire_gathers(0, 0), fire_gathers(1, 1), None]
    wcps = [None] * NSETS
    for k in range(NCHUNK):
        s = k % NSETS
        if k + 2 < NCHUNK:
            gcps[(k + 2) % NSETS] = fire_gathers(k + 2, (k + 2) % NSETS)
        with jax.named_scope("gwait"):
            for cp in gcps[s]:
                cp.wait()
        with jax.named_scope("wbwait"):
            if wcps[s] is not None:
                wcps[s].wait()
        b, o = bufs[s], obufs[s]

        def add_row(r, _, b=b, o=o):
            for col in range(D // LANES):
                sl = pl.ds(col * LANES, LANES)
                o[r, sl] = (b[r, sl] + b[r + CHUNK, sl]
                            + b[r + 2 * CHUNK, sl] + b[r + 3 * CHUNK, sl])
            return 0

        with jax.named_scope("adds"):
            lax.fori_loop(0, CHUNK, add_row, 0)
        wcps[s] = pltpu.async_copy(
            o, out.at[pl.ds(base + k * CHUNK, CHUNK)], wsems[s])
    for cp in wcps:
        if cp is not None:
            cp.wait()


def _entry(idx_hbm, table, out, idx_v, b0, b1, b2, o0, o1, o2, *sems):
    _body(idx_hbm, table, out, idx_v, b0, b1, b2, o0, o1, o2, sems)


_mesh = plsc.VectorSubcoreMesh(core_axis_name="c", subcore_axis_name="s")

_sc_call = functools.partial(
    pl.kernel,
    mesh=_mesh,
    out_type=jax.ShapeDtypeStruct((N, D), jnp.float32),
    scratch_types=(
        [pltpu.VMEM((2 * NCHUNK, 2 * CHUNK), jnp.int32)]
        + [pltpu.VMEM((GROWS, D), jnp.float32)] * NSETS
        + [pltpu.VMEM((CHUNK, D), jnp.float32)] * NSETS
        + [pltpu.SemaphoreType.DMA] * (2 * NSETS)
    ),
)(_entry)


@jax.jit
def kernel(coords, emb0, emb1, emb2, emb3):
    table = jnp.concatenate([emb0, emb1, emb2, emb3], axis=0)
    off = jnp.arange(LEVELS, dtype=jnp.int32) * emb0.shape[0]
    idx = (coords.reshape(NW, NCHUNK, CHUNK, LEVELS).transpose(0, 1, 3, 2)
           + off[None, None, :, None])
    idx = idx.reshape(NW, 2 * NCHUNK, 2 * CHUNK)
    return _sc_call(idx, table)
